# Initial kernel scaffold; baseline (speedup 1.0000x reference)
import jax
import jax.numpy as jnp
from jax.experimental import pallas as pl


def kernel(x, token_table, position_table):
    raise NotImplementedError("write your pallas kernel here")


# speedup vs baseline: 2.2589x; 2.2589x over previous
import functools

import jax
import jax.numpy as jnp
from jax import lax
from jax.experimental import pallas as pl
from jax.experimental.pallas import tpu as pltpu
from jax.experimental.pallas import tpu_sc as plsc

VOCAB = 1000000
D = 64
S = 200
B = 4096
N = B * S
NC, NS = 2, 16
NW = NC * NS
PER_W = N // NW
CHUNK = 2 * S
NCHUNK = PER_W // CHUNK
GATHER_SPLIT = 5
GSZ = CHUNK // GATHER_SPLIT


def _body(x_hbm, tok_hbm, pos_hbm, out_hbm, idx_v, rows_v, pos_v, sem):
    wid = lax.axis_index("s") * NC + lax.axis_index("c")
    base = wid * PER_W
    pltpu.sync_copy(pos_hbm, pos_v)

    def chunk_body(c, carry):
        cb = base + c * CHUNK
        pltpu.sync_copy(x_hbm.at[pl.ds(cb, CHUNK)], idx_v)
        cps = [
            pltpu.async_copy(
                tok_hbm.at[idx_v.at[pl.ds(j * GSZ, GSZ)]],
                rows_v.at[pl.ds(j * GSZ, GSZ)],
                sem,
            )
            for j in range(GATHER_SPLIT)
        ]
        for cp in cps:
            cp.wait()

        def add_body(r, carry2):
            for g in range(4):
                sl = pl.ds(g * 16, 16)
                p = pos_v[r, sl]
                rows_v[r, sl] += p
                rows_v[r + S, sl] += p
            return carry2

        lax.fori_loop(0, S, add_body, 0, unroll=2)
        pltpu.sync_copy(rows_v, out_hbm.at[pl.ds(cb, CHUNK)])
        return carry

    lax.fori_loop(0, NCHUNK, chunk_body, 0)


@jax.jit
def _embed(x_flat, token_table, position_table):
    mesh = plsc.VectorSubcoreMesh(core_axis_name="c", subcore_axis_name="s")
    k = functools.partial(
        pl.kernel,
        mesh=mesh,
        out_type=jax.ShapeDtypeStruct((N, D), jnp.float32),
        scratch_types=[
            pltpu.VMEM((CHUNK,), jnp.int32),
            pltpu.VMEM((CHUNK, D), jnp.float32),
            pltpu.VMEM((S, D), jnp.float32),
            pltpu.SemaphoreType.DMA,
        ],
        compiler_params=pltpu.CompilerParams(use_tc_tiling_on_sc=False),
    )(_body)
    return k(x_flat, token_table, position_table)


def kernel(x, token_table, position_table):
    x_flat = x.reshape(-1).astype(jnp.int32)
    out = _embed(x_flat, token_table, position_table)
    return out.reshape(B, S, D)
